# baseline (speedup 1.0000x reference)
import jax, jax.numpy as jnp
from jax.experimental import pallas as pl

N = 10000


def _matmul_pallas(a, w):
    def body(a_ref, w_ref, o_ref):
        o_ref[...] = jnp.dot(a_ref[...], w_ref[...],
                             preferred_element_type=jnp.float32)
    return pl.pallas_call(
        body,
        out_shape=jax.ShapeDtypeStruct((a.shape[0], w.shape[1]), jnp.float32),
    )(a, w)


def _dinv_pallas(deg):
    d2 = deg.reshape(1, N)
    def body(d_ref, o_ref):
        d = d_ref[...]
        o_ref[...] = jnp.where(d > 0, jax.lax.rsqrt(d), 0.0)
    return pl.pallas_call(
        body,
        out_shape=jax.ShapeDtypeStruct((1, N), jnp.float32),
    )(d2).reshape(N)


def _scale_rows_pallas(rows, scale):
    M, Hh = rows.shape
    BLK = 6600
    grid = M // BLK
    def body(r_ref, s_ref, o_ref):
        o_ref[...] = r_ref[...] * s_ref[...]
    return pl.pallas_call(
        body,
        grid=(grid,),
        in_specs=[pl.BlockSpec((BLK, Hh), lambda i: (i, 0)),
                  pl.BlockSpec((BLK, 1), lambda i: (i, 0))],
        out_specs=pl.BlockSpec((BLK, Hh), lambda i: (i, 0)),
        out_shape=jax.ShapeDtypeStruct((M, Hh), jnp.float32),
    )(rows, scale.reshape(M, 1))


def _heads_pallas(hm, Wc1, bc1, Wc2, bc2, Wr1, br1, Wr2, br2):
    def body(h_ref, wc1_ref, bc1_ref, wc2_ref, bc2_ref,
             wr1_ref, br1_ref, wr2_ref, br2_ref, cl_ref, rul_ref):
        h = h_ref[...]
        a = jax.nn.relu(jnp.dot(h, wc1_ref[...],
                                preferred_element_type=jnp.float32) + bc1_ref[...])
        cl_ref[...] = jnp.dot(a, wc2_ref[...],
                              preferred_element_type=jnp.float32) + bc2_ref[...]
        b = jax.nn.relu(jnp.dot(h, wr1_ref[...],
                                preferred_element_type=jnp.float32) + br1_ref[...])
        rul_ref[...] = jnp.dot(b, wr2_ref[...],
                               preferred_element_type=jnp.float32) + br2_ref[...]
    return pl.pallas_call(
        body,
        out_shape=(jax.ShapeDtypeStruct((1, Wc2.shape[1]), jnp.float32),
                   jax.ShapeDtypeStruct((1, Wr2.shape[1]), jnp.float32)),
    )(hm, Wc1, bc1, Wc2, bc2, Wr1, br1, Wr2, br2)


def _batchnorm(x, g, beta, eps=1e-5):
    mu = x.mean(axis=0)
    var = x.var(axis=0)
    return g * (x - mu) / jnp.sqrt(var + eps) + beta


def kernel(x, edge_index, W1, b1, g1, be1, W2, b2, g2, be2, W3, b3, g3, be3, Wc1, bc1, Wc2, bc2, Wr1, br1, Wr2, br2):
    src = edge_index[0]
    dst = edge_index[1]
    loop = jnp.arange(N, dtype=src.dtype)
    src2 = jnp.concatenate([src, loop])
    dst2 = jnp.concatenate([dst, loop])
    ones = jnp.ones(src2.shape[0], dtype=x.dtype)
    deg = jax.ops.segment_sum(ones, dst2, num_segments=N)
    dinv = _dinv_pallas(deg)
    norm = dinv[src2] * dinv[dst2]

    def conv(h, W, b):
        xw = _matmul_pallas(h, W)
        msg = _scale_rows_pallas(jnp.take(xw, src2, axis=0), norm)
        out = jax.ops.segment_sum(msg, dst2, num_segments=N)
        return out + b

    h = jax.nn.relu(_batchnorm(conv(x, W1, b1), g1, be1))
    h = jax.nn.relu(_batchnorm(conv(h, W2, b2), g2, be2))
    h = _batchnorm(conv(h, W3, b3), g3, be3)
    hm = h.mean(axis=0, keepdims=True)
    return _heads_pallas(hm, Wc1, bc1, Wc2, bc2, Wr1, br1, Wr2, br2)
